# baseline (speedup 1.0000x reference)
import jax
import jax.numpy as jnp
from jax import lax
from jax.experimental import pallas as pl
from jax.experimental.pallas import tpu as pltpu

TILE = 512


def _milnet_kernel(lo_ref, hi_ref, x_ref, w1_ref, b1_ref, w2_ref, b2_ref,
                   wv_ref, bv_ref, wu_ref, bu_ref, ww_ref,
                   wh1_ref, bh1_ref, wh2_ref, bh2_ref,
                   out_ref, m_ref, s_ref, z_ref):
    i = pl.program_id(0)
    n = pl.num_programs(0)

    @pl.when(i == 0)
    def _init():
        m_ref[...] = jnp.full_like(m_ref, -1e30)
        s_ref[...] = jnp.zeros_like(s_ref)
        z_ref[...] = jnp.zeros_like(z_ref)

    x = x_ref[...]
    h = jnp.dot(x, w1_ref[...], preferred_element_type=jnp.float32)
    h = jnp.maximum(h + b1_ref[...], 0.0)
    h = jnp.dot(h, w2_ref[...], preferred_element_type=jnp.float32)
    h = jnp.maximum(h + b2_ref[...], 0.0)
    av = jnp.tanh(jnp.dot(h, wv_ref[...], preferred_element_type=jnp.float32)
                  + bv_ref[...])
    au = jax.nn.sigmoid(jnp.dot(h, wu_ref[...], preferred_element_type=jnp.float32)
                        + bu_ref[...])
    scores = jnp.dot(av * au, ww_ref[...], preferred_element_type=jnp.float32)

    nb = m_ref.shape[1]
    gidx = i * TILE + lax.broadcasted_iota(jnp.int32, (TILE, 1), 0)
    mask = (gidx >= lo_ref[...]) & (gidx < hi_ref[...])
    ms = jnp.where(mask, scores, -1e30)
    tmax = jnp.max(ms, axis=0, keepdims=True)
    m_old = m_ref[...]
    m_new = jnp.maximum(m_old, tmax)
    alpha = jnp.exp(m_old - m_new)
    w = jnp.where(mask, jnp.exp(ms - m_new), 0.0)
    m_ref[...] = m_new
    s_ref[...] = s_ref[...] * alpha + jnp.sum(w, axis=0, keepdims=True)
    wz = lax.dot_general(w, h, (((0,), (0,)), ((), ())),
                         preferred_element_type=jnp.float32)
    z_ref[...] = z_ref[...] * alpha.reshape(nb, 1) + wz

    @pl.when(i == n - 1)
    def _finish():
        zm = z_ref[...] / s_ref[...].reshape(nb, 1)
        hh = jnp.dot(zm, wh1_ref[...], preferred_element_type=jnp.float32)
        hh = jnp.maximum(hh + bh1_ref[...], 0.0)
        out_ref[...] = (jnp.dot(hh, wh2_ref[...],
                                preferred_element_type=jnp.float32)
                        + bh2_ref[...])


def kernel(x_cells, bag_ptr, W1, b1, W2, b2, Wv, bv, Wu, bu, ww, Wh1, bh1,
           Wh2, bh2):
    total, in_dim = x_cells.shape
    nb = bag_ptr.shape[0] - 1
    ncls = Wh2.shape[1]
    grid = total // TILE

    lo = bag_ptr[:-1].reshape(1, nb).astype(jnp.int32)
    hi = bag_ptr[1:].reshape(1, nb).astype(jnp.int32)

    full = lambda a: pl.BlockSpec(a.shape, lambda i: (0,) * a.ndim)
    operands = (
        lo, hi, x_cells, W1, b1.reshape(1, -1), W2, b2.reshape(1, -1),
        Wv, bv.reshape(1, -1), Wu, bu.reshape(1, -1), ww,
        Wh1, bh1.reshape(1, -1), Wh2, bh2.reshape(1, -1),
    )
    in_specs = [full(a) for a in operands]
    in_specs[2] = pl.BlockSpec((TILE, in_dim), lambda i: (i, 0))

    out = pl.pallas_call(
        _milnet_kernel,
        grid=(grid,),
        in_specs=in_specs,
        out_specs=pl.BlockSpec((nb, ncls), lambda i: (0, 0)),
        out_shape=jax.ShapeDtypeStruct((nb, ncls), jnp.float32),
        scratch_shapes=[
            pltpu.VMEM((1, nb), jnp.float32),
            pltpu.VMEM((1, nb), jnp.float32),
            pltpu.VMEM((nb, Wh1.shape[0]), jnp.float32),
        ],
        compiler_params=pltpu.CompilerParams(
            dimension_semantics=("arbitrary",),
        ),
    )(*operands)
    return out


# speedup vs baseline: 1.0141x; 1.0141x over previous
import jax
import jax.numpy as jnp
from jax import lax
from jax.experimental import pallas as pl
from jax.experimental.pallas import tpu as pltpu

TILE = 512


def _milnet_kernel(lo_ref, hi_ref, x_ref, w1_ref, b1_ref, w2_ref, b2_ref,
                   wv_ref, bv_ref, wu_ref, bu_ref, ww_ref,
                   wh1_ref, bh1_ref, wh2_ref, bh2_ref,
                   out_ref, m_ref, s_ref, z_ref):
    i = pl.program_id(0)
    n = pl.num_programs(0)

    @pl.when(i == 0)
    def _init():
        m_ref[...] = jnp.full_like(m_ref, -1e30)
        s_ref[...] = jnp.zeros_like(s_ref)
        z_ref[...] = jnp.zeros_like(z_ref)

    x = x_ref[...].astype(jnp.bfloat16)
    h = jnp.dot(x, w1_ref[...].astype(jnp.bfloat16),
                preferred_element_type=jnp.float32)
    h = jnp.maximum(h + b1_ref[...], 0.0)
    h = jnp.dot(h.astype(jnp.bfloat16), w2_ref[...].astype(jnp.bfloat16),
                preferred_element_type=jnp.float32)
    h = jnp.maximum(h + b2_ref[...], 0.0)
    hb = h.astype(jnp.bfloat16)
    av = jnp.tanh(jnp.dot(hb, wv_ref[...].astype(jnp.bfloat16),
                          preferred_element_type=jnp.float32) + bv_ref[...])
    au = jax.nn.sigmoid(jnp.dot(hb, wu_ref[...].astype(jnp.bfloat16),
                                preferred_element_type=jnp.float32) + bu_ref[...])
    scores = jnp.dot(av * au, ww_ref[...], preferred_element_type=jnp.float32)

    nb = m_ref.shape[1]
    gidx = i * TILE + lax.broadcasted_iota(jnp.int32, (TILE, 1), 0)
    mask = (gidx >= lo_ref[...]) & (gidx < hi_ref[...])
    ms = jnp.where(mask, scores, -1e30)
    tmax = jnp.max(ms, axis=0, keepdims=True)
    m_old = m_ref[...]
    m_new = jnp.maximum(m_old, tmax)
    alpha = jnp.exp(m_old - m_new)
    w = jnp.where(mask, jnp.exp(ms - m_new), 0.0)
    m_ref[...] = m_new
    s_ref[...] = s_ref[...] * alpha + jnp.sum(w, axis=0, keepdims=True)
    wz = lax.dot_general(w, h, (((0,), (0,)), ((), ())),
                         preferred_element_type=jnp.float32)
    z_ref[...] = z_ref[...] * alpha.reshape(nb, 1) + wz

    @pl.when(i == n - 1)
    def _finish():
        zm = z_ref[...] / s_ref[...].reshape(nb, 1)
        hh = jnp.dot(zm, wh1_ref[...], preferred_element_type=jnp.float32)
        hh = jnp.maximum(hh + bh1_ref[...], 0.0)
        out_ref[...] = (jnp.dot(hh, wh2_ref[...],
                                preferred_element_type=jnp.float32)
                        + bh2_ref[...])


def kernel(x_cells, bag_ptr, W1, b1, W2, b2, Wv, bv, Wu, bu, ww, Wh1, bh1,
           Wh2, bh2):
    total, in_dim = x_cells.shape
    nb = bag_ptr.shape[0] - 1
    ncls = Wh2.shape[1]
    grid = total // TILE

    lo = bag_ptr[:-1].reshape(1, nb).astype(jnp.int32)
    hi = bag_ptr[1:].reshape(1, nb).astype(jnp.int32)

    full = lambda a: pl.BlockSpec(a.shape, lambda i: (0,) * a.ndim)
    operands = (
        lo, hi, x_cells, W1, b1.reshape(1, -1), W2, b2.reshape(1, -1),
        Wv, bv.reshape(1, -1), Wu, bu.reshape(1, -1), ww,
        Wh1, bh1.reshape(1, -1), Wh2, bh2.reshape(1, -1),
    )
    in_specs = [full(a) for a in operands]
    in_specs[2] = pl.BlockSpec((TILE, in_dim), lambda i: (i, 0))

    out = pl.pallas_call(
        _milnet_kernel,
        grid=(grid,),
        in_specs=in_specs,
        out_specs=pl.BlockSpec((nb, ncls), lambda i: (0, 0)),
        out_shape=jax.ShapeDtypeStruct((nb, ncls), jnp.float32),
        scratch_shapes=[
            pltpu.VMEM((1, nb), jnp.float32),
            pltpu.VMEM((1, nb), jnp.float32),
            pltpu.VMEM((nb, Wh1.shape[0]), jnp.float32),
        ],
        compiler_params=pltpu.CompilerParams(
            dimension_semantics=("arbitrary",),
        ),
    )(*operands)
    return out


# speedup vs baseline: 1.0827x; 1.0676x over previous
import jax
import jax.numpy as jnp
from jax import lax
from jax.experimental import pallas as pl
from jax.experimental.pallas import tpu as pltpu

TILE = 1024


def _milnet_kernel(lo_ref, hi_ref, x_ref, w1_ref, b1_ref, w2_ref, b2_ref,
                   wv_ref, bv_ref, wu_ref, bu_ref, ww_ref,
                   wh1_ref, bh1_ref, wh2_ref, bh2_ref,
                   out_ref, m_ref, s_ref, z_ref):
    i = pl.program_id(0)
    n = pl.num_programs(0)

    @pl.when(i == 0)
    def _init():
        m_ref[...] = jnp.full_like(m_ref, -1e30)
        s_ref[...] = jnp.zeros_like(s_ref)
        z_ref[...] = jnp.zeros_like(z_ref)

    x = x_ref[...].astype(jnp.bfloat16)
    h = jnp.dot(x, w1_ref[...].astype(jnp.bfloat16),
                preferred_element_type=jnp.float32)
    h = jnp.maximum(h + b1_ref[...], 0.0)
    h = jnp.dot(h.astype(jnp.bfloat16), w2_ref[...].astype(jnp.bfloat16),
                preferred_element_type=jnp.float32)
    h = jnp.maximum(h + b2_ref[...], 0.0)
    hb = h.astype(jnp.bfloat16)
    av = jnp.tanh(jnp.dot(hb, wv_ref[...].astype(jnp.bfloat16),
                          preferred_element_type=jnp.float32) + bv_ref[...])
    au = jax.nn.sigmoid(jnp.dot(hb, wu_ref[...].astype(jnp.bfloat16),
                                preferred_element_type=jnp.float32) + bu_ref[...])
    scores = jnp.dot(av * au, ww_ref[...], preferred_element_type=jnp.float32)

    nb = m_ref.shape[1]
    gidx = i * TILE + lax.broadcasted_iota(jnp.int32, (TILE, 1), 0)
    mask = (gidx >= lo_ref[...]) & (gidx < hi_ref[...])
    ms = jnp.where(mask, scores, -1e30)
    tmax = jnp.max(ms, axis=0, keepdims=True)
    m_old = m_ref[...]
    m_new = jnp.maximum(m_old, tmax)
    alpha = jnp.exp(m_old - m_new)
    w = jnp.where(mask, jnp.exp(ms - m_new), 0.0)
    m_ref[...] = m_new
    s_ref[...] = s_ref[...] * alpha + jnp.sum(w, axis=0, keepdims=True)
    wz = lax.dot_general(w, h, (((0,), (0,)), ((), ())),
                         preferred_element_type=jnp.float32)
    z_ref[...] = z_ref[...] * alpha.reshape(nb, 1) + wz

    @pl.when(i == n - 1)
    def _finish():
        zm = z_ref[...] / s_ref[...].reshape(nb, 1)
        hh = jnp.dot(zm, wh1_ref[...], preferred_element_type=jnp.float32)
        hh = jnp.maximum(hh + bh1_ref[...], 0.0)
        out_ref[...] = (jnp.dot(hh, wh2_ref[...],
                                preferred_element_type=jnp.float32)
                        + bh2_ref[...])


def kernel(x_cells, bag_ptr, W1, b1, W2, b2, Wv, bv, Wu, bu, ww, Wh1, bh1,
           Wh2, bh2):
    total, in_dim = x_cells.shape
    nb = bag_ptr.shape[0] - 1
    ncls = Wh2.shape[1]
    grid = total // TILE

    lo = bag_ptr[:-1].reshape(1, nb).astype(jnp.int32)
    hi = bag_ptr[1:].reshape(1, nb).astype(jnp.int32)

    full = lambda a: pl.BlockSpec(a.shape, lambda i: (0,) * a.ndim)
    operands = (
        lo, hi, x_cells, W1, b1.reshape(1, -1), W2, b2.reshape(1, -1),
        Wv, bv.reshape(1, -1), Wu, bu.reshape(1, -1), ww,
        Wh1, bh1.reshape(1, -1), Wh2, bh2.reshape(1, -1),
    )
    in_specs = [full(a) for a in operands]
    in_specs[2] = pl.BlockSpec((TILE, in_dim), lambda i: (i, 0))

    out = pl.pallas_call(
        _milnet_kernel,
        grid=(grid,),
        in_specs=in_specs,
        out_specs=pl.BlockSpec((nb, ncls), lambda i: (0, 0)),
        out_shape=jax.ShapeDtypeStruct((nb, ncls), jnp.float32),
        scratch_shapes=[
            pltpu.VMEM((1, nb), jnp.float32),
            pltpu.VMEM((1, nb), jnp.float32),
            pltpu.VMEM((nb, Wh1.shape[0]), jnp.float32),
        ],
        compiler_params=pltpu.CompilerParams(
            dimension_semantics=("arbitrary",),
        ),
    )(*operands)
    return out
